# baseline (speedup 1.0000x reference)
import functools

import numpy as np

import jax
import jax.numpy as jnp
from jax import lax
from jax.experimental import pallas as pl
from jax.experimental.pallas import tpu as pltpu
from jax.experimental.pallas import tpu_sc as plsc

N = 10000
E = 320000
D = 128
NC = 2
NS = 16
L = 16
CHUNK = 64
K = 320
G = 64
NBUF = 4
E_PAD = NS * CHUNK * K
NPAD = 10240
STRIPE = NPAD // NS
DUMMY = N
BLK = 1280


def _deg_body(dst_hbm, dinv_hbm, dstv, degv, stripev, dinvv, shared):
    c = lax.axis_index("c")
    s = lax.axis_index("s")
    pltpu.sync_copy(dst_hbm.at[c, s], dstv)

    zeros16 = jnp.zeros((L,), jnp.float32)
    ones16 = jnp.ones((L,), jnp.float32)

    @pl.loop(0, NPAD // L)
    def _(i):
        degv[pl.ds(i * L, L)] = zeros16

    @pl.loop(0, K)
    def _(j):
        @pl.loop(0, CHUNK // L)
        def _(i):
            idx = dstv[j, pl.ds(i * L, L)]
            plsc.addupdate_scatter(degv, [idx], ones16)

    pltpu.sync_copy(degv, shared.at[s])
    plsc.subcore_barrier()

    base = s * STRIPE
    pltpu.sync_copy(shared.at[:, pl.ds(base, STRIPE)], stripev)

    @pl.loop(0, STRIPE // L)
    def _(i):
        tot = stripev[0, pl.ds(i * L, L)]
        for r in range(1, NS):
            tot = tot + stripev[r, pl.ds(i * L, L)]
        d = tot + 1.0
        bits = plsc.bitcast(d, jnp.int32)
        bits = jnp.int32(0x5F3759DF) - (bits >> 1)
        y = plsc.bitcast(bits, jnp.float32)
        for _ in range(3):
            y = y * (1.5 - 0.5 * d * y * y)
        dinvv[pl.ds(i * L, L)] = y

    pltpu.sync_copy(dinvv, dinv_hbm.at[c, pl.ds(base, STRIPE)])


_deg_kernel = functools.partial(
    pl.kernel,
    compiler_params=pltpu.CompilerParams(needs_layout_passes=False),
    out_type=jax.ShapeDtypeStruct((NC, NPAD), jnp.float32),
    mesh=plsc.VectorSubcoreMesh(
        core_axis_name="c", subcore_axis_name="s", num_cores=NC, num_subcores=NS
    ),
    scratch_types=[
        pltpu.VMEM((K, CHUNK), jnp.int32),
        pltpu.VMEM((NPAD,), jnp.float32),
        pltpu.VMEM((NS, STRIPE), jnp.float32),
        pltpu.VMEM((STRIPE,), jnp.float32),
        pltpu.VMEM_SHARED((NS, NPAD), jnp.float32),
    ],
)(_deg_body)


def _edge_body(gp_hbm, src_hbm, dst_hbm, dinv_hbm, b_hbm, out_hbm,
               srcv, dstv, gb0, gb1, gb2, gb3, fb0, fb1, acc_sh, gsem, ssem):
    c = lax.axis_index("c")
    s = lax.axis_index("s")
    gbufs = (gb0, gb1, gb2, gb3)
    fbufs = (fb0, fb1)
    rows0, rows1 = fb0, fb1

    def gissue(j, b):
        pltpu.async_copy(gp_hbm.at[srcv.at[j]], gbufs[b], gsem)

    def gwait(b):
        pltpu.make_async_copy(gp_hbm.at[srcv.at[0]], gbufs[b], gsem).wait()

    def sissue(j, u):
        pltpu.async_copy(fbufs[u], acc_sh.at[dstv.at[j]], ssem, add=True)

    def swait(u):
        pltpu.make_async_copy(fbufs[u], acc_sh.at[dstv.at[0]], ssem).wait()

    def unpack_chunk(b, u):
        gb_, fb_ = gbufs[b], fbufs[u]

        @pl.loop(0, CHUNK)
        def _(r):
            for i in range(D // (2 * L)):
                w16 = gb_[r, pl.ds(i * L, L)]
                h2 = plsc.bitcast(w16, jnp.bfloat16)
                va, vb = plsc.unpack(h2, format=plsc.PackFormat.INTERLEAVED)
                fb_[r, pl.ds(2 * i * L, L)] = va
                fb_[r, pl.ds((2 * i + 1) * L, L)] = vb

    @pl.loop(0, STRIPE // CHUNK)
    def _(q):
        base = s * STRIPE + q * CHUNK
        pltpu.sync_copy(gp_hbm.at[pl.ds(c * NPAD + base, CHUNK)], gb0)
        unpack_chunk(0, 0)
        pltpu.sync_copy(fb0, acc_sh.at[pl.ds(base, CHUNK)])

    plsc.subcore_barrier()

    @pl.loop(0, K // G)
    def _(p):
        pltpu.sync_copy(src_hbm.at[c, s, pl.ds(p * G, G)], srcv)
        pltpu.sync_copy(dst_hbm.at[c, s, pl.ds(p * G, G)], dstv)

        @pl.when(p > 0)
        def _():
            swait(0)
            swait(1)

        for b in range(NBUF):
            gissue(b, b)

        @pl.loop(0, G, step=NBUF)
        def _(j):
            for b in range(NBUF):
                u = b % 2
                gwait(b)

                @pl.when(j + b >= 2)
                def _():
                    swait(u)
                unpack_chunk(b, u)
                sissue(j + b, u)

                @pl.when(j + b + NBUF < G)
                def _():
                    gissue(j + b + NBUF, b)

    swait(0)
    swait(1)
    plsc.subcore_barrier()

    pltpu.sync_copy(dinv_hbm.at[c, s], rows1.at[pl.ds(0, STRIPE // D)])
    pltpu.sync_copy(b_hbm, rows1.at[pl.ds(8, 1)])
    bvecs = [rows1[8, pl.ds(i * L, L)] for i in range(D // L)]

    @pl.loop(0, STRIPE // CHUNK)
    def _(q):
        base = s * STRIPE + q * CHUNK
        pltpu.sync_copy(acc_sh.at[pl.ds(base, CHUNK)], rows0)

        @pl.loop(0, CHUNK // L)
        def _(t):
            o = q * CHUNK + t * L
            dv = rows1[o // D, pl.ds(o % D, L)]
            for r in range(L):
                sc = jnp.take(dv, jnp.full((L,), r, jnp.int32))
                for i in range(D // L):
                    v = rows0[t * L + r, pl.ds(i * L, L)]
                    rows0[t * L + r, pl.ds(i * L, L)] = jnp.maximum(
                        v * sc + bvecs[i], 0.0)

        pltpu.sync_copy(rows0, out_hbm.at[c, pl.ds(base, CHUNK)])


_edge_kernel = functools.partial(
    pl.kernel,
    compiler_params=pltpu.CompilerParams(
        needs_layout_passes=False, use_tc_tiling_on_sc=False),
    out_type=jax.ShapeDtypeStruct((NC, NPAD, D), jnp.float32),
    mesh=plsc.VectorSubcoreMesh(
        core_axis_name="c", subcore_axis_name="s", num_cores=NC, num_subcores=NS
    ),
    scratch_types=[
        pltpu.VMEM((G, CHUNK), jnp.int32),
        pltpu.VMEM((G, CHUNK), jnp.int32),
        pltpu.VMEM((CHUNK, D // 2), jnp.int32),
        pltpu.VMEM((CHUNK, D // 2), jnp.int32),
        pltpu.VMEM((CHUNK, D // 2), jnp.int32),
        pltpu.VMEM((CHUNK, D // 2), jnp.int32),
        pltpu.VMEM((CHUNK, D), jnp.float32),
        pltpu.VMEM((CHUNK, D), jnp.float32),
        pltpu.VMEM_SHARED((NPAD, D), jnp.float32),
        pltpu.SemaphoreType.DMA,
        pltpu.SemaphoreType.DMA,
    ],
)(_edge_body)


def _mm_body(x_ref, d_ref, wa_ref, wb_ref, op_ref):
    x = x_ref[0]
    dv = d_ref[0]
    xsv = x * dv
    ga = jnp.dot(xsv, wa_ref[...], preferred_element_type=jnp.float32)
    gb = jnp.dot(xsv, wb_ref[...], preferred_element_type=jnp.float32)
    ai = lax.convert_element_type(
        lax.bitcast_convert_type(ga.astype(jnp.bfloat16), jnp.uint16),
        jnp.int32)
    bi = lax.convert_element_type(
        lax.bitcast_convert_type(gb.astype(jnp.bfloat16), jnp.uint16),
        jnp.int32)
    op_ref[0] = ai | (bi << 16)


def _matmul(xs, dinv_col, WA, WB):
    return pl.pallas_call(
        _mm_body,
        grid=(NC, NPAD // BLK),
        in_specs=[
            pl.BlockSpec((1, BLK, D), lambda g, j: (g, j, 0)),
            pl.BlockSpec((1, BLK, 1), lambda g, j: (g, j, 0)),
            pl.BlockSpec((D, D // 2), lambda g, j: (0, 0)),
            pl.BlockSpec((D, D // 2), lambda g, j: (0, 0)),
        ],
        out_specs=pl.BlockSpec((1, BLK, D // 2), lambda g, j: (g, j, 0)),
        out_shape=jax.ShapeDtypeStruct((NC, NPAD, D // 2), jnp.int32),
    )(xs, dinv_col, WA, WB)


def _fin_body(a_ref, g_ref, d_ref, b_ref, o_ref):
    o_ref[0] = jax.nn.relu((a_ref[0] + g_ref[0]) * d_ref[0] + b_ref[...])


def _finalize(acc, g, dinv_col, b2d):
    return pl.pallas_call(
        _fin_body,
        grid=(NC, NPAD // BLK),
        in_specs=[
            pl.BlockSpec((1, BLK, D), lambda g, j: (g, j, 0)),
            pl.BlockSpec((1, BLK, D), lambda g, j: (g, j, 0)),
            pl.BlockSpec((1, BLK, 1), lambda g, j: (g, j, 0)),
            pl.BlockSpec((1, D), lambda g, j: (0, 0)),
        ],
        out_specs=pl.BlockSpec((1, BLK, D), lambda g, j: (g, j, 0)),
        out_shape=jax.ShapeDtypeStruct((NC, NPAD, D), jnp.float32),
    )(acc, g, dinv_col, b2d)


def _prep_edges(ei, src_off):
    pad = jnp.full((E_PAD - E,), DUMMY, jnp.int32)
    src = jnp.concatenate([ei[0], pad]).reshape(NS, K, CHUNK) + src_off
    dst = jnp.concatenate([ei[1], pad]).reshape(NS, K, CHUNK)
    return src, dst


def kernel(x1, edge_index1, x2, edge_index2, W, b):
    s1, d1 = _prep_edges(edge_index1, 0)
    s2, d2 = _prep_edges(edge_index2, NPAD)
    src_all = jnp.stack([s1, s2])
    dst_all = jnp.stack([d1, d2])

    dinv = _deg_kernel(dst_all)
    dinv_col = dinv[:, :, None]

    xs = jnp.pad(jnp.stack([x1, x2]), ((0, 0), (0, NPAD - N), (0, 0)))
    perm_a = np.concatenate([np.arange(32 * i, 32 * i + 16) for i in range(4)])
    gp = _matmul(xs, dinv_col, W[:, perm_a], W[:, perm_a + 16])

    dinv4 = dinv.reshape(NC, NS, STRIPE // D, D)
    y = _edge_kernel(gp.reshape(NC * NPAD, D // 2),
                     src_all, dst_all, dinv4, b.reshape(1, D))
    return (y[0, :N], y[1, :N])


# speedup vs baseline: 1.1157x; 1.1157x over previous
import functools

import numpy as np

import jax
import jax.numpy as jnp
from jax import lax
from jax.experimental import pallas as pl
from jax.experimental.pallas import tpu as pltpu
from jax.experimental.pallas import tpu_sc as plsc

N = 10000
E = 320000
D = 128
NC = 2
NS = 16
L = 16
CHUNK = 64
K = 320
G = 64
NBUF = 4
E_PAD = NS * CHUNK * K
NPAD = 10240
STRIPE = NPAD // NS
DUMMY = N
BLK = 1280


def _deg_body(dst_hbm, dinv_hbm, dstv, degv, stripev, dinvv, shared):
    c = lax.axis_index("c")
    s = lax.axis_index("s")
    pltpu.sync_copy(dst_hbm.at[c, s], dstv)

    zeros16 = jnp.zeros((L,), jnp.float32)
    ones16 = jnp.ones((L,), jnp.float32)

    @pl.loop(0, NPAD // L)
    def _(i):
        degv[pl.ds(i * L, L)] = zeros16

    @pl.loop(0, K)
    def _(j):
        @pl.loop(0, CHUNK // L)
        def _(i):
            idx = dstv[j, pl.ds(i * L, L)]
            plsc.addupdate_scatter(degv, [idx], ones16)

    pltpu.sync_copy(degv, shared.at[s])
    plsc.subcore_barrier()

    base = s * STRIPE
    pltpu.sync_copy(shared.at[:, pl.ds(base, STRIPE)], stripev)

    @pl.loop(0, STRIPE // L)
    def _(i):
        tot = stripev[0, pl.ds(i * L, L)]
        for r in range(1, NS):
            tot = tot + stripev[r, pl.ds(i * L, L)]
        d = tot + 1.0
        bits = plsc.bitcast(d, jnp.int32)
        bits = jnp.int32(0x5F3759DF) - (bits >> 1)
        y = plsc.bitcast(bits, jnp.float32)
        for _ in range(3):
            y = y * (1.5 - 0.5 * d * y * y)
        dinvv[pl.ds(i * L, L)] = y

    pltpu.sync_copy(dinvv, dinv_hbm.at[c, pl.ds(base, STRIPE)])


_deg_kernel = functools.partial(
    pl.kernel,
    compiler_params=pltpu.CompilerParams(needs_layout_passes=False),
    out_type=jax.ShapeDtypeStruct((NC, NPAD), jnp.float32),
    mesh=plsc.VectorSubcoreMesh(
        core_axis_name="c", subcore_axis_name="s", num_cores=NC, num_subcores=NS
    ),
    scratch_types=[
        pltpu.VMEM((K, CHUNK), jnp.int32),
        pltpu.VMEM((NPAD,), jnp.float32),
        pltpu.VMEM((NS, STRIPE), jnp.float32),
        pltpu.VMEM((STRIPE,), jnp.float32),
        pltpu.VMEM_SHARED((NS, NPAD), jnp.float32),
    ],
)(_deg_body)


def _edge_body(g_hbm, gp_hbm, src_hbm, dst_hbm, dinv_hbm, b_hbm, out_hbm,
               srcv, dstv, gb0, gb1, gb2, gb3, fb0, fb1, acc_sh, gsem, ssem):
    c = lax.axis_index("c")
    s = lax.axis_index("s")
    gbufs = (gb0, gb1, gb2, gb3)
    fbufs = (fb0, fb1)
    rows0, rows1, rows2 = fb0, fb1, gb0

    pltpu.sync_copy(
        g_hbm.at[pl.ds(c * NPAD + s * STRIPE, STRIPE)],
        acc_sh.at[pl.ds(s * STRIPE, STRIPE)],
    )
    plsc.subcore_barrier()

    def gissue(j, b):
        pltpu.async_copy(gp_hbm.at[srcv.at[j]], gbufs[b], gsem)

    def gwait(b):
        pltpu.make_async_copy(gp_hbm.at[srcv.at[0]], gbufs[b], gsem).wait()

    def sissue(j, u):
        pltpu.async_copy(fbufs[u], acc_sh.at[dstv.at[j]], ssem, add=True)

    def swait(u):
        pltpu.make_async_copy(fbufs[u], acc_sh.at[dstv.at[0]], ssem).wait()

    def unpack_chunk(b, u):
        gb_, fb_ = gbufs[b], fbufs[u]

        @pl.loop(0, CHUNK)
        def _(r):
            for i in range(D // (2 * L)):
                w16 = gb_[r, pl.ds(i * L, L)]
                h2 = plsc.bitcast(w16, jnp.bfloat16)
                va, vb = plsc.unpack(h2, format=plsc.PackFormat.INTERLEAVED)
                fb_[r, pl.ds(2 * i * L, L)] = va
                fb_[r, pl.ds((2 * i + 1) * L, L)] = vb

    @pl.loop(0, K // G)
    def _(p):
        pltpu.sync_copy(src_hbm.at[c, s, pl.ds(p * G, G)], srcv)
        pltpu.sync_copy(dst_hbm.at[c, s, pl.ds(p * G, G)], dstv)

        @pl.when(p > 0)
        def _():
            swait(0)
            swait(1)

        for b in range(NBUF):
            gissue(b, b)

        @pl.loop(0, G, step=NBUF)
        def _(j):
            for b in range(NBUF):
                u = b % 2
                gwait(b)

                @pl.when(j + b >= 2)
                def _():
                    swait(u)
                unpack_chunk(b, u)
                sissue(j + b, u)

                @pl.when(j + b + NBUF < G)
                def _():
                    gissue(j + b + NBUF, b)

    swait(0)
    swait(1)
    plsc.subcore_barrier()

    pltpu.sync_copy(dinv_hbm.at[c, s], rows1.at[pl.ds(0, STRIPE // D)])
    pltpu.sync_copy(b_hbm, rows1.at[pl.ds(8, 1)])
    bvecs = [rows1[8, pl.ds(i * L, L)] for i in range(D // L)]

    @pl.loop(0, STRIPE // CHUNK)
    def _(q):
        base = s * STRIPE + q * CHUNK
        pltpu.sync_copy(acc_sh.at[pl.ds(base, CHUNK)], rows0)

        @pl.loop(0, CHUNK // L)
        def _(t):
            o = q * CHUNK + t * L
            dv = rows1[o // D, pl.ds(o % D, L)]
            for r in range(L):
                sc = jnp.take(dv, jnp.full((L,), r, jnp.int32))
                for i in range(D // L):
                    v = rows0[t * L + r, pl.ds(i * L, L)]
                    rows0[t * L + r, pl.ds(i * L, L)] = jnp.maximum(
                        v * sc + bvecs[i], 0.0)

        pltpu.sync_copy(rows0, out_hbm.at[c, pl.ds(base, CHUNK)])


_edge_kernel = functools.partial(
    pl.kernel,
    compiler_params=pltpu.CompilerParams(
        needs_layout_passes=False, use_tc_tiling_on_sc=False),
    out_type=jax.ShapeDtypeStruct((NC, NPAD, D), jnp.float32),
    mesh=plsc.VectorSubcoreMesh(
        core_axis_name="c", subcore_axis_name="s", num_cores=NC, num_subcores=NS
    ),
    scratch_types=[
        pltpu.VMEM((G, CHUNK), jnp.int32),
        pltpu.VMEM((G, CHUNK), jnp.int32),
        pltpu.VMEM((CHUNK, D // 2), jnp.int32),
        pltpu.VMEM((CHUNK, D // 2), jnp.int32),
        pltpu.VMEM((CHUNK, D // 2), jnp.int32),
        pltpu.VMEM((CHUNK, D // 2), jnp.int32),
        pltpu.VMEM((CHUNK, D), jnp.float32),
        pltpu.VMEM((CHUNK, D), jnp.float32),
        pltpu.VMEM_SHARED((NPAD, D), jnp.float32),
        pltpu.SemaphoreType.DMA,
        pltpu.SemaphoreType.DMA,
    ],
)(_edge_body)


def _mm_body(x_ref, d_ref, w_ref, wa_ref, wb_ref, o_ref, op_ref):
    x = x_ref[0]
    dv = d_ref[0]
    xsv = x * dv
    o_ref[0] = jnp.dot(xsv, w_ref[...], preferred_element_type=jnp.float32)
    ga = jnp.dot(xsv, wa_ref[...], preferred_element_type=jnp.float32)
    gb = jnp.dot(xsv, wb_ref[...], preferred_element_type=jnp.float32)
    ai = lax.convert_element_type(
        lax.bitcast_convert_type(ga.astype(jnp.bfloat16), jnp.uint16),
        jnp.int32)
    bi = lax.convert_element_type(
        lax.bitcast_convert_type(gb.astype(jnp.bfloat16), jnp.uint16),
        jnp.int32)
    op_ref[0] = ai | (bi << 16)


def _matmul(xs, dinv_col, W, WA, WB):
    return pl.pallas_call(
        _mm_body,
        grid=(NC, NPAD // BLK),
        in_specs=[
            pl.BlockSpec((1, BLK, D), lambda g, j: (g, j, 0)),
            pl.BlockSpec((1, BLK, 1), lambda g, j: (g, j, 0)),
            pl.BlockSpec((D, D), lambda g, j: (0, 0)),
            pl.BlockSpec((D, D // 2), lambda g, j: (0, 0)),
            pl.BlockSpec((D, D // 2), lambda g, j: (0, 0)),
        ],
        out_specs=[
            pl.BlockSpec((1, BLK, D), lambda g, j: (g, j, 0)),
            pl.BlockSpec((1, BLK, D // 2), lambda g, j: (g, j, 0)),
        ],
        out_shape=[
            jax.ShapeDtypeStruct((NC, NPAD, D), jnp.float32),
            jax.ShapeDtypeStruct((NC, NPAD, D // 2), jnp.int32),
        ],
    )(xs, dinv_col, W, WA, WB)


def _fin_body(a_ref, g_ref, d_ref, b_ref, o_ref):
    o_ref[0] = jax.nn.relu((a_ref[0] + g_ref[0]) * d_ref[0] + b_ref[...])


def _finalize(acc, g, dinv_col, b2d):
    return pl.pallas_call(
        _fin_body,
        grid=(NC, NPAD // BLK),
        in_specs=[
            pl.BlockSpec((1, BLK, D), lambda g, j: (g, j, 0)),
            pl.BlockSpec((1, BLK, D), lambda g, j: (g, j, 0)),
            pl.BlockSpec((1, BLK, 1), lambda g, j: (g, j, 0)),
            pl.BlockSpec((1, D), lambda g, j: (0, 0)),
        ],
        out_specs=pl.BlockSpec((1, BLK, D), lambda g, j: (g, j, 0)),
        out_shape=jax.ShapeDtypeStruct((NC, NPAD, D), jnp.float32),
    )(acc, g, dinv_col, b2d)


def _prep_edges(ei, src_off):
    pad = jnp.full((E_PAD - E,), DUMMY, jnp.int32)
    src = jnp.concatenate([ei[0], pad]).reshape(NS, K, CHUNK) + src_off
    dst = jnp.concatenate([ei[1], pad]).reshape(NS, K, CHUNK)
    return src, dst


def kernel(x1, edge_index1, x2, edge_index2, W, b):
    s1, d1 = _prep_edges(edge_index1, 0)
    s2, d2 = _prep_edges(edge_index2, NPAD)
    src_all = jnp.stack([s1, s2])
    dst_all = jnp.stack([d1, d2])

    dinv = _deg_kernel(dst_all)
    dinv_col = dinv[:, :, None]

    xs = jnp.pad(jnp.stack([x1, x2]), ((0, 0), (0, NPAD - N), (0, 0)))
    perm_a = np.concatenate([np.arange(32 * i, 32 * i + 16) for i in range(4)])
    g, gp = _matmul(xs, dinv_col, W, W[:, perm_a], W[:, perm_a + 16])

    dinv4 = dinv.reshape(NC, NS, STRIPE // D, D)
    y = _edge_kernel(g.reshape(NC * NPAD, D), gp.reshape(NC * NPAD, D // 2),
                     src_all, dst_all, dinv4, b.reshape(1, D))
    return (y[0, :N], y[1, :N])
